# baseline (speedup 1.0000x reference)
import functools

import jax
import jax.numpy as jnp
from jax import lax
from jax.experimental import pallas as pl
from jax.experimental.pallas import tpu as pltpu
from jax.experimental.pallas import tpu_sc as plsc

N = 10000
E = 320000
D = 64
G = 64
NC = 2
NS = 16
NW = NC * NS
EPW = E // NW
K = 125
C = EPW // K
NBUF = 8
DEGW = 16
NPD = 10240
DPT = NPD // NS
NPA = 10240
RPT = NPA // NS

_mesh = plsc.VectorSubcoreMesh(core_axis_name="c", subcore_axis_name="s")


@functools.partial(
    pl.kernel,
    out_type=jax.ShapeDtypeStruct((NC * NPD, DEGW), jnp.float32),
    mesh=_mesh,
    scratch_types=[
        pltpu.VMEM((C, K), jnp.int32),
        pltpu.VMEM((K, DEGW), jnp.float32),
        pltpu.VMEM_SHARED((NPD, DEGW), jnp.float32),
    ],
    compiler_params=pltpu.CompilerParams(use_tc_tiling_on_sc=False),
)
def _sc_degree(dst3_hbm, ones_hbm, zdeg_hbm, out_hbm, didx_v, ones_v, acc_sh):
    cid = lax.axis_index("c")
    sid = lax.axis_index("s")
    wid = cid * NS + sid
    pltpu.sync_copy(zdeg_hbm, acc_sh.at[pl.ds(sid * DPT, DPT)])
    pltpu.sync_copy(ones_hbm, ones_v)
    pltpu.sync_copy(dst3_hbm.at[wid], didx_v)
    plsc.subcore_barrier()

    @pl.loop(0, C)
    def _chunk(j):
        pltpu.sync_copy(ones_v, acc_sh.at[didx_v.at[j]], add=True)

    plsc.subcore_barrier()
    pltpu.sync_copy(acc_sh.at[pl.ds(sid * DPT, DPT)],
                    out_hbm.at[pl.ds(cid * NPD + sid * DPT, DPT)])


@functools.partial(
    pl.kernel,
    out_type=jax.ShapeDtypeStruct((NC * NPA, D), jnp.float32),
    mesh=_mesh,
    scratch_types=[
        pltpu.VMEM((C, K), jnp.int32),
        pltpu.VMEM((C, K), jnp.int32),
        pltpu.VMEM((NBUF, K, D), jnp.float32),
        [pltpu.SemaphoreType.DMA] * NBUF,
        [pltpu.SemaphoreType.DMA] * NBUF,
        pltpu.VMEM_SHARED((NPA, D), jnp.float32),
    ],
    compiler_params=pltpu.CompilerParams(use_tc_tiling_on_sc=False),
)
def _sc_scatter(t_hbm, src3_hbm, dst3_hbm, zacc_hbm, out_hbm,
                sidx_v, didx_v, rows_v, gsems, ssems, acc_sh):
    cid = lax.axis_index("c")
    sid = lax.axis_index("s")
    wid = cid * NS + sid
    pltpu.sync_copy(zacc_hbm, acc_sh.at[pl.ds(sid * RPT, RPT)])
    pltpu.sync_copy(src3_hbm.at[wid], sidx_v)
    pltpu.sync_copy(dst3_hbm.at[wid], didx_v)
    plsc.subcore_barrier()

    for b in range(NBUF):
        pltpu.async_copy(t_hbm.at[sidx_v.at[b]], rows_v.at[b], gsems[b])

    @pl.loop(0, C, step=NBUF)
    def _ring(j):
        for b in range(NBUF):
            jj = j + b
            pltpu.make_async_copy(t_hbm.at[sidx_v.at[0]], rows_v.at[b],
                                  gsems[b]).wait()
            pltpu.async_copy(rows_v.at[b], acc_sh.at[didx_v.at[jj]], ssems[b],
                             add=True)
            pltpu.make_async_copy(rows_v.at[b], acc_sh.at[didx_v.at[0]],
                                  ssems[b]).wait()

            @pl.when(jj + NBUF < C)
            def _prefetch():
                pltpu.async_copy(t_hbm.at[sidx_v.at[jj + NBUF]], rows_v.at[b],
                                 gsems[b])

    plsc.subcore_barrier()
    pltpu.sync_copy(acc_sh.at[pl.ds(sid * RPT, RPT)],
                    out_hbm.at[pl.ds(cid * NPA + sid * RPT, RPT)])


def _bn(o, g, be):
    mu = jnp.mean(o, axis=0, keepdims=True)
    var = jnp.mean((o - mu) * (o - mu), axis=0, keepdims=True)
    return (o - mu) * lax.rsqrt(var + 1e-5) * g + be


def _tc_prep_body(x_ref, w1_ref, degp_ref, t1_ref, dinv_ref):
    deg = degp_ref[0:N, 0:1] + degp_ref[NPD:NPD + N, 0:1] + 1.0
    dinv = lax.rsqrt(deg)
    h = jnp.dot(x_ref[...], w1_ref[...], preferred_element_type=jnp.float32)
    t1_ref[...] = h * dinv
    dinv_ref[...] = dinv


def _tc_layer_body(sp_ref, t_ref, dinv_ref, b_ref, g_ref, be_ref, w_ref,
                   out_ref, *, relu):
    dinv = dinv_ref[...]
    s = sp_ref[0:N, :] + sp_ref[NPA:NPA + N, :] + t_ref[...]
    o = s * dinv + b_ref[...]
    if relu:
        o = jnp.maximum(o, 0.0)
    h = _bn(o, g_ref[...], be_ref[...])
    out_ref[...] = jnp.dot(h, w_ref[...],
                           preferred_element_type=jnp.float32) * dinv


def _tc_final_body(sp_ref, t_ref, dinv_ref, b_ref, g_ref, be_ref, batch_ref,
                   wm0_ref, bm0_ref, gm0_ref, bem0_ref,
                   wm1_ref, bm1_ref, gm1_ref, bem1_ref,
                   wo_ref, bo_ref, out_ref):
    dinv = dinv_ref[...]
    s = sp_ref[0:N, :] + sp_ref[NPA:NPA + N, :] + t_ref[...]
    h = _bn(s * dinv + b_ref[...], g_ref[...], be_ref[...])
    gids = lax.broadcasted_iota(jnp.int32, (G, N), 0)
    onehot = (gids == batch_ref[...]).astype(jnp.float32)
    p = jnp.dot(onehot, h, preferred_element_type=jnp.float32)
    p = jnp.maximum(jnp.dot(p, wm0_ref[...],
                            preferred_element_type=jnp.float32) + bm0_ref[...], 0.0)
    p = _bn(p, gm0_ref[...], bem0_ref[...])
    p = jnp.maximum(jnp.dot(p, wm1_ref[...],
                            preferred_element_type=jnp.float32) + bm1_ref[...], 0.0)
    p = _bn(p, gm1_ref[...], bem1_ref[...])
    out_ref[...] = jnp.dot(p, wo_ref[...],
                           preferred_element_type=jnp.float32) + bo_ref[...]


_f32 = jnp.float32

_tc_prep = pl.pallas_call(
    _tc_prep_body,
    out_shape=(jax.ShapeDtypeStruct((N, D), _f32),
               jax.ShapeDtypeStruct((N, 1), _f32)),
)

_tc_layer_relu = pl.pallas_call(
    functools.partial(_tc_layer_body, relu=True),
    out_shape=jax.ShapeDtypeStruct((N, D), _f32),
)

_tc_layer = pl.pallas_call(
    functools.partial(_tc_layer_body, relu=False),
    out_shape=jax.ShapeDtypeStruct((N, D), _f32),
)

_tc_final = pl.pallas_call(
    _tc_final_body,
    out_shape=jax.ShapeDtypeStruct((G, 1), _f32),
)


def kernel(x, edge_index, batch, W1, b1, g1, be1, Wc, bc, gc, bec,
           Wm, bm, gm, bem, Wo, bo):
    src3 = edge_index[0].reshape(NW, C, K)
    dst3 = edge_index[1].reshape(NW, C, K)
    ones = jnp.ones((K, DEGW), _f32)
    z_deg = jnp.zeros((DPT, DEGW), _f32)
    z_acc = jnp.zeros((RPT, D), _f32)
    batch2 = batch.reshape(1, N)

    degp = _sc_degree(dst3, ones, z_deg)
    t1, dinv = _tc_prep(x, W1, degp)

    s = _sc_scatter(t1, src3, dst3, z_acc)
    t2 = _tc_layer_relu(s, t1, dinv, b1.reshape(1, D), g1.reshape(1, D),
                        be1.reshape(1, D), Wc[0])
    s = _sc_scatter(t2, src3, dst3, z_acc)
    t3 = _tc_layer(s, t2, dinv, bc[0].reshape(1, D), gc[0].reshape(1, D),
                   bec[0].reshape(1, D), Wc[1])
    s = _sc_scatter(t3, src3, dst3, z_acc)
    t4 = _tc_layer(s, t3, dinv, bc[1].reshape(1, D), gc[1].reshape(1, D),
                   bec[1].reshape(1, D), Wc[2])
    s = _sc_scatter(t4, src3, dst3, z_acc)
    return _tc_final(s, t4, dinv, bc[2].reshape(1, D), gc[2].reshape(1, D),
                     bec[2].reshape(1, D), batch2,
                     Wm[0], bm[0].reshape(1, D), gm[0].reshape(1, D),
                     bem[0].reshape(1, D),
                     Wm[1], bm[1].reshape(1, D), gm[1].reshape(1, D),
                     bem[1].reshape(1, D),
                     Wo, bo.reshape(1, 1))


# speedup vs baseline: 1.1874x; 1.1874x over previous
import functools

import jax
import jax.numpy as jnp
from jax import lax
from jax.experimental import pallas as pl
from jax.experimental.pallas import tpu as pltpu
from jax.experimental.pallas import tpu_sc as plsc

N = 10000
E = 320000
D = 64
G = 64
NC = 2
NS = 16
NW = NC * NS
EPW = E // NW
K = 125
C = EPW // K
NBUF = 4
DEGW = 16
NPD = 10240
DPT = NPD // NS
NPA = 10240
RPT = NPA // NS
L = N // 2
PA2 = NPA // 2

_mesh = plsc.VectorSubcoreMesh(core_axis_name="c", subcore_axis_name="s")


@functools.partial(
    pl.kernel,
    out_type=jax.ShapeDtypeStruct((NC * NPD, DEGW), jnp.float32),
    mesh=_mesh,
    scratch_types=[
        pltpu.VMEM((C, K), jnp.int32),
        pltpu.VMEM((K, DEGW), jnp.float32),
        pltpu.VMEM_SHARED((NPD, DEGW), jnp.float32),
    ],
    compiler_params=pltpu.CompilerParams(use_tc_tiling_on_sc=False),
)
def _sc_degree(dst3_hbm, ones_hbm, zdeg_hbm, out_hbm, didx_v, ones_v, acc_sh):
    cid = lax.axis_index("c")
    sid = lax.axis_index("s")
    wid = cid * NS + sid
    pltpu.sync_copy(zdeg_hbm, acc_sh.at[pl.ds(sid * DPT, DPT)])
    pltpu.sync_copy(ones_hbm, ones_v)
    pltpu.sync_copy(dst3_hbm.at[wid], didx_v)
    plsc.subcore_barrier()

    @pl.loop(0, C)
    def _chunk(j):
        pltpu.sync_copy(ones_v, acc_sh.at[didx_v.at[j]], add=True)

    plsc.subcore_barrier()
    pltpu.sync_copy(acc_sh.at[pl.ds(sid * DPT, DPT)],
                    out_hbm.at[pl.ds(cid * NPD + sid * DPT, DPT)])


@functools.partial(
    pl.kernel,
    out_type=jax.ShapeDtypeStruct((NC * NPA, D), jnp.float32),
    mesh=_mesh,
    scratch_types=[
        pltpu.VMEM((C, K), jnp.int32),
        pltpu.VMEM((C, K), jnp.int32),
        pltpu.VMEM((NBUF, K, D), jnp.float32),
        [pltpu.SemaphoreType.DMA] * NBUF,
        [pltpu.SemaphoreType.DMA] * NBUF,
        pltpu.VMEM_SHARED((NPA, D), jnp.float32),
    ],
    compiler_params=pltpu.CompilerParams(use_tc_tiling_on_sc=False),
)
def _sc_scatter(t_hbm, src3_hbm, dst3_hbm, zacc_hbm, out_hbm,
                sidx_v, didx_v, rows_v, gsems, ssems, acc_sh):
    cid = lax.axis_index("c")
    sid = lax.axis_index("s")
    wid = cid * NS + sid
    pltpu.sync_copy(zacc_hbm, acc_sh.at[pl.ds(sid * RPT, RPT)])
    pltpu.sync_copy(src3_hbm.at[wid], sidx_v)
    pltpu.sync_copy(dst3_hbm.at[wid], didx_v)
    plsc.subcore_barrier()

    for b in range(NBUF):
        pltpu.async_copy(t_hbm.at[sidx_v.at[b]], rows_v.at[b], gsems[b])

    @pl.loop(0, C, step=NBUF)
    def _ring(j):
        for b in range(NBUF):
            jj = j + b
            pltpu.make_async_copy(t_hbm.at[sidx_v.at[0]], rows_v.at[b],
                                  gsems[b]).wait()
            pltpu.async_copy(rows_v.at[b], acc_sh.at[didx_v.at[jj]], ssems[b],
                             add=True)
            pltpu.make_async_copy(rows_v.at[b], acc_sh.at[didx_v.at[0]],
                                  ssems[b]).wait()

            @pl.when(jj + NBUF < C)
            def _prefetch():
                pltpu.async_copy(t_hbm.at[sidx_v.at[jj + NBUF]], rows_v.at[b],
                                 gsems[b])

    plsc.subcore_barrier()
    pltpu.sync_copy(acc_sh.at[pl.ds(sid * RPT, RPT)],
                    out_hbm.at[pl.ds(cid * NPA + sid * RPT, RPT)])


def _bn(o, g, be):
    mu = jnp.mean(o, axis=0, keepdims=True)
    var = jnp.mean((o - mu) * (o - mu), axis=0, keepdims=True)
    return (o - mu) / jnp.sqrt(var + 1e-5) * g + be


def _pack_cols(a):
    m = 0.5 * (a[:, 0:D] + a[:, D:2 * D])
    return jnp.concatenate([m, m], axis=1)


def _bn_packed(o, g, be):
    mu = _pack_cols(jnp.mean(o, axis=0, keepdims=True))
    c = o - mu
    var = _pack_cols(jnp.mean(c * c, axis=0, keepdims=True))
    return c / jnp.sqrt(var + 1e-5) * g + be


def _tc_prep_body(x_ref, w1_ref, degp_ref, t1_ref, dinvp_ref):
    deg_lo = degp_ref[0:L, 0:1] + degp_ref[NPD:NPD + L, 0:1]
    deg_hi = degp_ref[L:N, 0:1] + degp_ref[NPD + L:NPD + N, 0:1]
    dinv2 = 1.0 / jnp.sqrt(jnp.concatenate([deg_lo, deg_hi], axis=1) + 1.0)
    dinvp = jnp.concatenate(
        [jnp.broadcast_to(dinv2[:, 0:1], (L, D)),
         jnp.broadcast_to(dinv2[:, 1:2], (L, D))], axis=1)
    h = jnp.dot(x_ref[...], w1_ref[...], preferred_element_type=jnp.float32)
    hp = jnp.concatenate([h[0:L, :], h[L:N, :]], axis=1)
    t1_ref[...] = hp * dinvp
    dinvp_ref[...] = dinvp


def _tc_layer_body(sp_ref, t_ref, dinvp_ref, b_ref, g_ref, be_ref, w2_ref,
                   out_ref, *, relu):
    dinvp = dinvp_ref[...]
    s = sp_ref[0:L, :] + sp_ref[PA2:PA2 + L, :] + t_ref[...]
    o = s * dinvp + b_ref[...]
    if relu:
        o = jnp.maximum(o, 0.0)
    h = _bn_packed(o, g_ref[...], be_ref[...])
    out_ref[...] = jnp.dot(h, w2_ref[...],
                           preferred_element_type=jnp.float32) * dinvp


def _tc_final_body(sp_ref, t_ref, dinvp_ref, b_ref, g_ref, be_ref,
                   blo_ref, bhi_ref,
                   wm0_ref, bm0_ref, gm0_ref, bem0_ref,
                   wm1_ref, bm1_ref, gm1_ref, bem1_ref,
                   wo_ref, bo_ref, out_ref):
    dinvp = dinvp_ref[...]
    s = sp_ref[0:L, :] + sp_ref[PA2:PA2 + L, :] + t_ref[...]
    h = _bn_packed(s * dinvp + b_ref[...], g_ref[...], be_ref[...])
    gids = lax.broadcasted_iota(jnp.int32, (G, L), 0)
    oh_lo = (gids == blo_ref[...]).astype(jnp.float32)
    oh_hi = (gids == bhi_ref[...]).astype(jnp.float32)
    p = (jnp.dot(oh_lo, h[:, 0:D], preferred_element_type=jnp.float32,
                 precision=lax.Precision.HIGHEST)
         + jnp.dot(oh_hi, h[:, D:2 * D], preferred_element_type=jnp.float32,
                   precision=lax.Precision.HIGHEST))
    p = jnp.maximum(jnp.dot(p, wm0_ref[...],
                            preferred_element_type=jnp.float32) + bm0_ref[...], 0.0)
    p = _bn(p, gm0_ref[...], bem0_ref[...])
    p = jnp.maximum(jnp.dot(p, wm1_ref[...],
                            preferred_element_type=jnp.float32) + bm1_ref[...], 0.0)
    p = _bn(p, gm1_ref[...], bem1_ref[...])
    out_ref[...] = jnp.dot(p, wo_ref[...],
                           preferred_element_type=jnp.float32) + bo_ref[...]


_f32 = jnp.float32

_tc_prep = pl.pallas_call(
    _tc_prep_body,
    out_shape=(jax.ShapeDtypeStruct((L, 2 * D), _f32),
               jax.ShapeDtypeStruct((L, 2 * D), _f32)),
)

_tc_layer_relu = pl.pallas_call(
    functools.partial(_tc_layer_body, relu=True),
    out_shape=jax.ShapeDtypeStruct((L, 2 * D), _f32),
)

_tc_layer = pl.pallas_call(
    functools.partial(_tc_layer_body, relu=False),
    out_shape=jax.ShapeDtypeStruct((L, 2 * D), _f32),
)

_tc_final = pl.pallas_call(
    _tc_final_body,
    out_shape=jax.ShapeDtypeStruct((G, 1), _f32),
)


def _tile2(v):
    return jnp.concatenate([v, v]).reshape(1, 2 * D)


def _blockdiag2(w):
    z = jnp.zeros((D, D), _f32)
    return jnp.concatenate([jnp.concatenate([w, z], axis=1),
                            jnp.concatenate([z, w], axis=1)], axis=0)


def kernel(x, edge_index, batch, W1, b1, g1, be1, Wc, bc, gc, bec,
           Wm, bm, gm, bem, Wo, bo):
    src = edge_index[0]
    dst = edge_index[1]
    srcp3 = (2 * (src % L) + src // L).reshape(NW, C, K)
    dstp3 = (2 * (dst % L) + dst // L).reshape(NW, C, K)
    dst3 = dst.reshape(NW, C, K)
    ones = jnp.ones((K, DEGW), _f32)
    z_deg = jnp.zeros((DPT, DEGW), _f32)
    z_acc = jnp.zeros((RPT, D), _f32)

    degp = _sc_degree(dst3, ones, z_deg)
    t1, dinvp = _tc_prep(x, W1, degp)

    def scatter(tp):
        s = _sc_scatter(tp.reshape(N, D), srcp3, dstp3, z_acc)
        return s.reshape(2 * PA2, 2 * D)

    t2 = _tc_layer_relu(scatter(t1), t1, dinvp, _tile2(b1), _tile2(g1),
                        _tile2(be1), _blockdiag2(Wc[0]))
    t3 = _tc_layer(scatter(t2), t2, dinvp, _tile2(bc[0]), _tile2(gc[0]),
                   _tile2(bec[0]), _blockdiag2(Wc[1]))
    t4 = _tc_layer(scatter(t3), t3, dinvp, _tile2(bc[1]), _tile2(gc[1]),
                   _tile2(bec[1]), _blockdiag2(Wc[2]))
    return _tc_final(scatter(t4), t4, dinvp, _tile2(bc[2]), _tile2(gc[2]),
                     _tile2(bec[2]),
                     batch[0:L].reshape(1, L), batch[L:N].reshape(1, L),
                     Wm[0], bm[0].reshape(1, D), gm[0].reshape(1, D),
                     bem[0].reshape(1, D),
                     Wm[1], bm[1].reshape(1, D), gm[1].reshape(1, D),
                     bem[1].reshape(1, D),
                     Wo, bo.reshape(1, 1))
